# baseline (speedup 1.0000x reference)
import functools

import jax
import jax.numpy as jnp
from jax import lax
from jax.experimental import pallas as pl
from jax.experimental.pallas import tpu as pltpu
from jax.experimental.pallas import tpu_sc as plsc


def _sc_gather_tiled(table, idxp, n_rowtiles):
    V, D = table.shape
    RT = n_rowtiles
    info = plsc.get_sparse_core_info()
    NC, NS = info.num_cores, info.num_subcores
    NW = NC * NS
    assert RT % NW == 0
    rt_per_w = RT // NW
    idx_per_w = rt_per_w * 416

    mesh = plsc.VectorSubcoreMesh(core_axis_name="c", subcore_axis_name="s")

    @functools.partial(
        pl.kernel,
        mesh=mesh,
        out_type=jax.ShapeDtypeStruct((RT * 104, 128), table.dtype),
        scratch_types=[
            pltpu.VMEM((idx_per_w,), jnp.int32),
            pltpu.VMEM((2, 4, 104, 32), table.dtype),
            pltpu.SemaphoreType.DMA((2,)),
            pltpu.SemaphoreType.DMA((2,)),
        ],
        compiler_params=pltpu.CompilerParams(use_tc_tiling_on_sc=False),
    )
    def k(table_hbm, idx_hbm, out_hbm, idx_v, rows_v, gsem, ssem):
        wid = lax.axis_index("s") * NC + lax.axis_index("c")
        pltpu.sync_copy(idx_hbm.at[pl.ds(wid * idx_per_w, idx_per_w)], idx_v)

        def start_gathers(c):
            slot = c % 2
            copies = []
            for j in range(4):
                copies.append(
                    pltpu.async_copy(
                        table_hbm.at[idx_v.at[pl.ds(c * 416 + j * 104, 104)]],
                        rows_v.at[slot, j],
                        gsem.at[slot],
                    )
                )
            return copies

        def start_scatters(c):
            slot = c % 2
            base = (wid * rt_per_w + c) * 104
            copies = []
            for j in range(4):
                copies.append(
                    pltpu.async_copy(
                        rows_v.at[slot, j],
                        out_hbm.at[pl.ds(base, 104), pl.ds(j * 32, 32)],
                        ssem.at[slot],
                    )
                )
            return copies

        gathers = [start_gathers(0)]
        scatters = [None, None]
        for c in range(rt_per_w):
            for g in gathers[c]:
                g.wait()
            scatters[c % 2] = start_scatters(c)
            if c + 1 < rt_per_w:
                if scatters[(c + 1) % 2] is not None:
                    for s in scatters[(c + 1) % 2]:
                        s.wait()
                gathers.append(start_gathers(c + 1))
        for sc in scatters:
            if sc is not None:
                for s in sc:
                    s.wait()

    return k(table, idxp)


def _mlp13_body(nct, x_ref, w1_ref, b1_ref, w2_ref, b2_ref, o_ref):
    nb = o_ref.shape[0]
    x4 = x_ref[...].reshape(nb // 8, nct, 8, 128)
    acc = jnp.dot(
        x4[:, 0].reshape(nb, 128), w1_ref[0], preferred_element_type=jnp.float32
    )
    for c in range(1, nct):
        acc = acc + jnp.dot(
            x4[:, c].reshape(nb, 128), w1_ref[c],
            preferred_element_type=jnp.float32,
        )
    h = jnp.maximum(acc + b1_ref[...], 0.0)
    z = jnp.dot(h, w2_ref[...], preferred_element_type=jnp.float32) + b2_ref[...]
    z = z - jnp.max(z, axis=-1, keepdims=True)
    e = jnp.exp(z)
    o_ref[...] = e / jnp.sum(e, axis=-1, keepdims=True)


def _mlp13(xq, B, W1r, b1, W2, b2, block_b=512, interpret=False):
    nct, K, H = W1r.shape
    _, O = W2.shape
    nblk = B // block_b
    return pl.pallas_call(
        functools.partial(_mlp13_body, nct),
        grid=(nblk,),
        in_specs=[
            pl.BlockSpec((block_b * nct, 128), lambda i: (i, 0)),
            pl.BlockSpec((nct, K, H), lambda i: (0, 0, 0)),
            pl.BlockSpec((1, H), lambda i: (0, 0)),
            pl.BlockSpec((H, O), lambda i: (0, 0)),
            pl.BlockSpec((1, O), lambda i: (0, 0)),
        ],
        out_specs=pl.BlockSpec((block_b, O), lambda i: (i, 0)),
        out_shape=jax.ShapeDtypeStruct((B, O), jnp.float32),
        interpret=interpret,
    )(xq, W1r, b1.reshape(1, -1), W2, b2.reshape(1, -1))


def kernel(inputs, table, W1, b1, W2, b2):
    B, S = inputs.shape
    V, D = table.shape
    S2 = ((S + 3) // 4) * 4
    nct = (S2 * D) // 128
    idx52 = jnp.pad(inputs.astype(jnp.int32), ((0, 0), (0, S2 - S)))
    A = idx52.reshape(B // 8, 8, nct, 4)
    idxp = A.transpose(0, 3, 2, 1).reshape(-1)
    xq = _sc_gather_tiled(table, idxp, B // 8)
    W1r = jnp.pad(W1, ((0, S2 * D - S * D), (0, 0))).reshape(nct, 128, -1)
    return _mlp13(xq, B, W1r, b1, W2, b2)
